# baseline (speedup 1.0000x reference)
import math

import jax
import jax.numpy as jnp
from jax.experimental import pallas as pl
from jax.experimental.pallas import tpu as pltpu

_BLK = 512
_HALF_LOG_2PI = 0.5 * math.log(2.0 * math.pi)


def _body(betas_ref, gumbel_ref, eps_ref, muss_ref, stdss_ref, acts_ref, lp_ref):
    blk, kk, aa = muss_ref.shape
    betas = betas_ref[...]
    logits = jnp.log(betas / jnp.sum(betas, axis=-1, keepdims=True))
    score = logits + gumbel_ref[...]

    best = score[:, 0:1]
    comp = jnp.zeros((blk, 1), jnp.int32)
    for k in range(1, kk):
        sk = score[:, k : k + 1]
        upd = sk > best
        comp = jnp.where(upd, k, comp)
        best = jnp.where(upd, sk, best)

    mus = muss_ref[...]
    stds = stdss_ref[...]

    kiota = jax.lax.broadcasted_iota(jnp.int32, (blk, kk, 1), 1)
    sel = comp[:, :, None] == kiota
    mu_sel = jnp.sum(jnp.where(sel, mus, 0.0), axis=1)
    std_sel = jnp.sum(jnp.where(sel, stds, 0.0), axis=1)

    acts = jnp.clip(mu_sel + std_sel * eps_ref[...], -1.0, 1.0)
    acts_ref[...] = acts

    z = (acts[:, None, :] - mus) / stds
    log_comp = jnp.sum(-0.5 * z * z - jnp.log(stds), axis=-1)
    x = logits + log_comp - (aa * _HALF_LOG_2PI)
    m = jnp.max(x, axis=-1, keepdims=True)
    lp = jnp.log(jnp.sum(jnp.exp(x - m), axis=-1, keepdims=True)) + m
    lp_ref[...] = lp


def kernel(muss, stdss, betas):
    b, k, a = muss.shape
    kc, kn = jax.random.split(jax.random.key(42))
    gumbel = jax.random.gumbel(kc, (b, k), muss.dtype)
    eps = jax.random.normal(kn, (b, a), muss.dtype)

    grid = (b // _BLK,)
    acts, lp = pl.pallas_call(
        _body,
        grid=grid,
        in_specs=[
            pl.BlockSpec((_BLK, k), lambda i: (i, 0)),
            pl.BlockSpec((_BLK, k), lambda i: (i, 0)),
            pl.BlockSpec((_BLK, a), lambda i: (i, 0)),
            pl.BlockSpec((_BLK, k, a), lambda i: (i, 0, 0)),
            pl.BlockSpec((_BLK, k, a), lambda i: (i, 0, 0)),
        ],
        out_specs=[
            pl.BlockSpec((_BLK, a), lambda i: (i, 0)),
            pl.BlockSpec((_BLK, 1), lambda i: (i, 0)),
        ],
        out_shape=[
            jax.ShapeDtypeStruct((b, a), muss.dtype),
            jax.ShapeDtypeStruct((b, 1), muss.dtype),
        ],
    )(betas, gumbel, eps, muss, stdss)
    return acts, lp.reshape(b)
